# baseline (speedup 1.0000x reference)
import numpy as np
import jax
import jax.numpy as jnp
from jax.experimental import pallas as pl

N = 4096
D_IN = 128
RANK = 16
D_HIDDEN = 128
BOTTLENECK = 64
TOPK = 32

PRE_BLK = 512
BLK = 512

_SIGN = np.int32(-2**31)
_MANT = np.int32(0x7FFFFFFF)


def _keys(l):
    bits = jax.lax.bitcast_convert_type(l, jnp.int32)
    return jnp.where(bits < 0, bits ^ _MANT, bits)


def _unkey(k):
    bits = jnp.where(k < 0, k ^ _MANT, k)
    return jax.lax.bitcast_convert_type(bits, jnp.float32)


def _prelude_kernel(x_ref, ew1_ref, eb1_ref, ew2_ref, eb2_ref, ws_ref, wr_ref,
                    b_ref, xm_ref, sb_ref, r_ref):
    x = x_ref[...]
    h = jnp.maximum(
        jnp.dot(x, ew1_ref[...], preferred_element_type=jnp.float32)
        + eb1_ref[...], 0.0)
    xm = x + jnp.dot(h, ew2_ref[...], preferred_element_type=jnp.float32) \
        + eb2_ref[...]
    xm_ref[...] = xm
    s = jax.nn.softmax(
        jnp.dot(xm, ws_ref[...], preferred_element_type=jnp.float32), axis=-1)
    r = jax.nn.softmax(
        jnp.dot(xm, wr_ref[...], preferred_element_type=jnp.float32), axis=-1)
    sb_ref[...] = jnp.dot(s, b_ref[...], preferred_element_type=jnp.float32)
    r_ref[...] = r


def _kth_key(vals, k, lo, hi):
    def cond(carry):
        lo_c, hi_c = carry
        return jnp.any(hi_c > lo_c)

    def step(carry):
        lo_c, hi_c = carry
        c = lo_c + jax.lax.shift_right_logical((hi_c - lo_c) + 1, 1)
        cnt = jnp.sum((vals >= _unkey(c)).astype(jnp.int32), axis=1,
                      keepdims=True)
        feas = cnt >= k
        return jnp.where(feas, c, lo_c), jnp.where(feas, hi_c, c - 1)

    def body(carry):
        for _ in range(4):
            carry = step(carry)
        return carry

    lo, hi = jax.lax.while_loop(cond, body, (lo, hi))
    return lo


def _bisect16(count_fn, k, lo, hi, unroll):
    def cond(carry):
        lo_c, hi_c = carry
        return jnp.any(hi_c > lo_c)

    def step(carry):
        lo_c, hi_c = carry
        c = lo_c + jax.lax.shift_right_logical((hi_c - lo_c) + 1, 1)
        feas = count_fn(c.astype(jnp.int16)) >= jnp.float32(k)
        return jnp.where(feas, c, lo_c), jnp.where(feas, hi_c, c - 1)

    def body(carry):
        for _ in range(unroll):
            carry = step(carry)
        return carry

    lo, hi = jax.lax.while_loop(cond, body, (lo, hi))
    return lo


def _count_cols(mask_bf16, ones_col):
    return jax.lax.dot_general(mask_bf16, ones_col, (((1,), (0,)), ((), ())),
                               preferred_element_type=jnp.float32)


def _kth_key_split(l, k, t0, hi1):
    ki = _keys(l)
    khi = jax.lax.shift_right_arithmetic(ki, 16).astype(jnp.int16)
    klo = ((ki & jnp.int32(0xFFFF)) ^ jnp.int32(0x8000)).astype(jnp.int16)
    one = jnp.bfloat16(1)
    zero = jnp.bfloat16(0)
    ones_col = jnp.full((l.shape[1], 1), 1, jnp.bfloat16)

    def cnt_hi(c16):
        return _count_cols(jnp.where(khi >= c16, one, zero), ones_col)

    t_hi = _bisect16(cnt_hi, k,
                     jax.lax.shift_right_arithmetic(t0, 16),
                     jax.lax.shift_right_arithmetic(hi1, 16), unroll=4)
    t_hi16 = t_hi.astype(jnp.int16)
    n_gt = _count_cols(jnp.where(khi > t_hi16, one, zero), ones_col)
    eq = jnp.where(khi == t_hi16, one, zero)

    def cnt_lo(c16):
        return n_gt + _count_cols(jnp.where(klo >= c16, eq, zero), ones_col)

    full = jnp.full_like(t_hi, 0)
    c_lo = _bisect16(cnt_lo, k, full - 32768, full + 32767, unroll=8)
    return jax.lax.shift_left(t_hi, 16) | \
        ((c_lo & jnp.int32(0xFFFF)) ^ jnp.int32(0x8000))


def _round1_kernel(sb_ref, r_ref, xm_ref, gw1_ref, gb1_ref,
                   h1_ref, m_ref, emin_ref, s_ref):
    sb = sb_ref[...]
    r = r_ref[...]
    l = jax.lax.dot_general(sb, r, (((1,), (1,)), ((), ())),
                            preferred_element_type=jnp.float32)

    m = jnp.max(l, axis=1, keepdims=True)
    lo1 = _keys(jnp.min(l, axis=1, keepdims=True))
    hi1 = _keys(m)
    t_star = _kth_key_split(l, TOPK, lo1, hi1)
    thr_f = _unkey(t_star)

    e = jnp.exp(l - m)
    z = jnp.sum(e, axis=1, keepdims=True)
    e_thr = jnp.exp(thr_f - m)
    a_thr = e_thr / z
    bits_t = jax.lax.bitcast_convert_type(e_thr, jnp.int32)
    e_min = e_thr
    for d in range(1, 9):
        cand = jax.lax.bitcast_convert_type(
            jnp.maximum(bits_t - d, 0), jnp.float32)
        ok = (cand / z) >= a_thr
        e_min = jnp.where(ok, cand, e_min)
    kept = e >= e_min
    ek = jnp.where(kept, e, 0.0)
    sume = jnp.sum(ek, axis=1, keepdims=True)
    rz = 1.0 / z
    den2 = jnp.maximum(sume * rz, 1e-8)
    s = rz * (1.0 / den2)
    w = ek * s
    msg = jnp.dot(w, xm_ref[...], preferred_element_type=jnp.float32)
    h1 = jnp.maximum(
        jnp.dot(msg, gw1_ref[...], preferred_element_type=jnp.float32)
        + gb1_ref[...], 0.0)
    h1_ref[...] = h1
    m_ref[...] = m
    emin_ref[...] = e_min
    s_ref[...] = s


def _round2_kernel(sb_ref, r_ref, h1_ref, m_ref, emin_ref, s_ref,
                   gw2_ref, gb2_ref, gwo_ref, gbo_ref, y_ref):
    sb = sb_ref[...]
    r = r_ref[...]
    l = jax.lax.dot_general(sb, r, (((1,), (1,)), ((), ())),
                            preferred_element_type=jnp.float32)
    e = jnp.exp(l - m_ref[...])
    w = jnp.where(e >= emin_ref[...], e, 0.0) * s_ref[...]
    msg = jnp.dot(w, h1_ref[...], preferred_element_type=jnp.float32)
    h2 = jnp.maximum(
        jnp.dot(msg, gw2_ref[...], preferred_element_type=jnp.float32)
        + gb2_ref[...], 0.0)
    y_ref[...] = jnp.dot(h2, gwo_ref[...], preferred_element_type=jnp.float32) \
        + gbo_ref[...]


def kernel(X, enc_W1, enc_b1, enc_W2, enc_b2, Ws, Wr, B,
           g_W1, g_b1, g_W2, g_b2, g_Wo, g_bo):
    eb1 = enc_b1.reshape(1, BOTTLENECK)
    eb2 = enc_b2.reshape(1, D_IN)
    gb1 = g_b1.reshape(1, D_HIDDEN)
    gb2 = g_b2.reshape(1, D_HIDDEN)
    gbo = g_bo.reshape(1, 1)

    full = lambda shape: pl.BlockSpec(shape, lambda i: (0, 0))

    xm, sb, r = pl.pallas_call(
        _prelude_kernel,
        grid=(N // PRE_BLK,),
        in_specs=[
            pl.BlockSpec((PRE_BLK, D_IN), lambda i: (i, 0)),
            full((D_IN, BOTTLENECK)), full((1, BOTTLENECK)),
            full((BOTTLENECK, D_IN)), full((1, D_IN)),
            full((D_IN, RANK)), full((D_IN, RANK)), full((RANK, RANK)),
        ],
        out_specs=[
            pl.BlockSpec((PRE_BLK, D_IN), lambda i: (i, 0)),
            pl.BlockSpec((PRE_BLK, RANK), lambda i: (i, 0)),
            pl.BlockSpec((PRE_BLK, RANK), lambda i: (i, 0)),
        ],
        out_shape=[
            jax.ShapeDtypeStruct((N, D_IN), jnp.float32),
            jax.ShapeDtypeStruct((N, RANK), jnp.float32),
            jax.ShapeDtypeStruct((N, RANK), jnp.float32),
        ],
    )(X, enc_W1, eb1, enc_W2, eb2, Ws, Wr, B)

    rowspec = pl.BlockSpec((BLK, 1), lambda i: (i, 0))
    rowshape = jax.ShapeDtypeStruct((N, 1), jnp.float32)
    h1, mrow, emin, srow = pl.pallas_call(
        _round1_kernel,
        grid=(N // BLK,),
        in_specs=[
            pl.BlockSpec((BLK, RANK), lambda i: (i, 0)),
            full((N, RANK)), full((N, D_IN)),
            full((D_IN, D_HIDDEN)), full((1, D_HIDDEN)),
        ],
        out_specs=[
            pl.BlockSpec((BLK, D_HIDDEN), lambda i: (i, 0)),
            rowspec, rowspec, rowspec,
        ],
        out_shape=[
            jax.ShapeDtypeStruct((N, D_HIDDEN), jnp.float32),
            rowshape, rowshape, rowshape,
        ],
    )(sb, r, xm, g_W1, gb1)

    y = pl.pallas_call(
        _round2_kernel,
        grid=(N // BLK,),
        in_specs=[
            pl.BlockSpec((BLK, RANK), lambda i: (i, 0)),
            full((N, RANK)), full((N, D_HIDDEN)),
            rowspec, rowspec, rowspec,
            full((D_HIDDEN, D_HIDDEN)), full((1, D_HIDDEN)),
            full((D_HIDDEN, 1)), full((1, 1)),
        ],
        out_specs=pl.BlockSpec((BLK, 1), lambda i: (i, 0)),
        out_shape=jax.ShapeDtypeStruct((N, 1), jnp.float32),
    )(sb, r, h1, mrow, emin, srow, g_W2, gb2, g_Wo, gbo)

    return y


# speedup vs baseline: 1.8440x; 1.8440x over previous
import numpy as np
import jax
import jax.numpy as jnp
from jax.experimental import pallas as pl

N = 4096
D_IN = 128
RANK = 16
D_HIDDEN = 128
BOTTLENECK = 64
TOPK = 32

PRE_BLK = 512
BLK = 512

_SIGN = np.int32(-2**31)
_MANT = np.int32(0x7FFFFFFF)


def _keys(l):
    bits = jax.lax.bitcast_convert_type(l, jnp.int32)
    return jnp.where(bits < 0, bits ^ _MANT, bits)


def _unkey(k):
    bits = jnp.where(k < 0, k ^ _MANT, k)
    return jax.lax.bitcast_convert_type(bits, jnp.float32)


def _prelude_kernel(x_ref, ew1_ref, eb1_ref, ew2_ref, eb2_ref, ws_ref, wr_ref,
                    b_ref, xm_ref, sb_ref, r_ref):
    x = x_ref[...]
    h = jnp.maximum(
        jnp.dot(x, ew1_ref[...], preferred_element_type=jnp.float32)
        + eb1_ref[...], 0.0)
    xm = x + jnp.dot(h, ew2_ref[...], preferred_element_type=jnp.float32) \
        + eb2_ref[...]
    xm_ref[...] = xm
    s = jax.nn.softmax(
        jnp.dot(xm, ws_ref[...], preferred_element_type=jnp.float32), axis=-1)
    r = jax.nn.softmax(
        jnp.dot(xm, wr_ref[...], preferred_element_type=jnp.float32), axis=-1)
    sb_ref[...] = jnp.dot(s, b_ref[...], preferred_element_type=jnp.float32)
    r_ref[...] = r


def _kth_key(vals, k, lo, hi):
    def cond(carry):
        lo_c, hi_c = carry
        return jnp.any(hi_c > lo_c)

    def step(carry):
        lo_c, hi_c = carry
        c = lo_c + jax.lax.shift_right_logical((hi_c - lo_c) + 1, 1)
        cnt = jnp.sum((vals >= _unkey(c)).astype(jnp.int32), axis=1,
                      keepdims=True)
        feas = cnt >= k
        return jnp.where(feas, c, lo_c), jnp.where(feas, hi_c, c - 1)

    def body(carry):
        for _ in range(4):
            carry = step(carry)
        return carry

    lo, hi = jax.lax.while_loop(cond, body, (lo, hi))
    return lo


def _bisect16(count_fn, k, lo, hi, unroll):
    def cond(carry):
        lo_c, hi_c = carry
        return jnp.any(hi_c > lo_c)

    def step(carry):
        lo_c, hi_c = carry
        c = lo_c + jax.lax.shift_right_logical((hi_c - lo_c) + 1, 1)
        feas = count_fn(c.astype(jnp.int16)) >= k
        return jnp.where(feas, c, lo_c), jnp.where(feas, hi_c, c - 1)

    def body(carry):
        for _ in range(unroll):
            carry = step(carry)
        return carry

    lo, hi = jax.lax.while_loop(cond, body, (lo, hi))
    return lo


def _sum16(x):
    w = x.shape[1]
    while w > 256:
        w //= 2
        x = x[:, :w] + x[:, w:2 * w]
    return jnp.sum(x.astype(jnp.int32), axis=1, keepdims=True)


def _kth_key_split(l, k, t0, hi1):
    ki = _keys(l)
    khi = jax.lax.shift_right_arithmetic(ki, 16).astype(jnp.int16)
    klo = ((ki & jnp.int32(0xFFFF)) ^ jnp.int32(0x8000)).astype(jnp.int16)
    one = jnp.int16(1)
    zero = jnp.int16(0)

    def cnt_hi(c16):
        return _sum16(jnp.where(khi >= c16, one, zero))

    t_hi = _bisect16(cnt_hi, k,
                     jax.lax.shift_right_arithmetic(t0, 16),
                     jax.lax.shift_right_arithmetic(hi1, 16), unroll=4)
    t_hi16 = t_hi.astype(jnp.int16)
    n_gt = _sum16(jnp.where(khi > t_hi16, one, zero))
    eq = jnp.where(khi == t_hi16, one, zero)

    def cnt_lo(c16):
        return n_gt + _sum16(jnp.where(klo >= c16, eq, zero))

    lo_c = jnp.full_like(t_hi, -32768)
    hi_c = jnp.full_like(t_hi, 32767)
    for _ in range(16):
        c = lo_c + jax.lax.shift_right_logical((hi_c - lo_c) + 1, 1)
        feas = cnt_lo(c.astype(jnp.int16)) >= k
        lo_c = jnp.where(feas, c, lo_c)
        hi_c = jnp.where(feas, hi_c, c - 1)
    c_lo = lo_c
    return jax.lax.shift_left(t_hi, 16) | \
        ((c_lo & jnp.int32(0xFFFF)) ^ jnp.int32(0x8000))


def _round1_kernel(sb_ref, r_ref, xm_ref, gw1_ref, gb1_ref,
                   h1_ref, m_ref, emin_ref, s_ref):
    sb = sb_ref[...]
    r = r_ref[...]
    l = jax.lax.dot_general(sb, r, (((1,), (1,)), ((), ())),
                            preferred_element_type=jnp.float32)

    m = jnp.max(l, axis=1, keepdims=True)
    lo1 = _keys(jnp.min(l, axis=1, keepdims=True))
    hi1 = _keys(m)
    t_star = _kth_key_split(l, TOPK, lo1, hi1)
    thr_f = _unkey(t_star)

    e = jnp.exp(l - m)
    z = jnp.sum(e, axis=1, keepdims=True)
    e_thr = jnp.exp(thr_f - m)
    a_thr = e_thr / z
    bits_t = jax.lax.bitcast_convert_type(e_thr, jnp.int32)
    e_min = e_thr
    for d in range(1, 9):
        cand = jax.lax.bitcast_convert_type(
            jnp.maximum(bits_t - d, 0), jnp.float32)
        ok = (cand / z) >= a_thr
        e_min = jnp.where(ok, cand, e_min)
    kept = e >= e_min
    ek = jnp.where(kept, e, 0.0)
    sume = jnp.sum(ek, axis=1, keepdims=True)
    rz = 1.0 / z
    den2 = jnp.maximum(sume * rz, 1e-8)
    s = rz * (1.0 / den2)
    w = ek * s
    msg = jnp.dot(w, xm_ref[...], preferred_element_type=jnp.float32)
    h1 = jnp.maximum(
        jnp.dot(msg, gw1_ref[...], preferred_element_type=jnp.float32)
        + gb1_ref[...], 0.0)
    h1_ref[...] = h1
    m_ref[...] = m
    emin_ref[...] = e_min
    s_ref[...] = s


def _round2_kernel(sb_ref, r_ref, h1_ref, m_ref, emin_ref, s_ref,
                   gw2_ref, gb2_ref, gwo_ref, gbo_ref, y_ref):
    sb = sb_ref[...]
    r = r_ref[...]
    l = jax.lax.dot_general(sb, r, (((1,), (1,)), ((), ())),
                            preferred_element_type=jnp.float32)
    e = jnp.exp(l - m_ref[...])
    w = jnp.where(e >= emin_ref[...], e, 0.0) * s_ref[...]
    msg = jnp.dot(w, h1_ref[...], preferred_element_type=jnp.float32)
    h2 = jnp.maximum(
        jnp.dot(msg, gw2_ref[...], preferred_element_type=jnp.float32)
        + gb2_ref[...], 0.0)
    y_ref[...] = jnp.dot(h2, gwo_ref[...], preferred_element_type=jnp.float32) \
        + gbo_ref[...]


def kernel(X, enc_W1, enc_b1, enc_W2, enc_b2, Ws, Wr, B,
           g_W1, g_b1, g_W2, g_b2, g_Wo, g_bo):
    eb1 = enc_b1.reshape(1, BOTTLENECK)
    eb2 = enc_b2.reshape(1, D_IN)
    gb1 = g_b1.reshape(1, D_HIDDEN)
    gb2 = g_b2.reshape(1, D_HIDDEN)
    gbo = g_bo.reshape(1, 1)

    full = lambda shape: pl.BlockSpec(shape, lambda i: (0, 0))

    xm, sb, r = pl.pallas_call(
        _prelude_kernel,
        grid=(N // PRE_BLK,),
        in_specs=[
            pl.BlockSpec((PRE_BLK, D_IN), lambda i: (i, 0)),
            full((D_IN, BOTTLENECK)), full((1, BOTTLENECK)),
            full((BOTTLENECK, D_IN)), full((1, D_IN)),
            full((D_IN, RANK)), full((D_IN, RANK)), full((RANK, RANK)),
        ],
        out_specs=[
            pl.BlockSpec((PRE_BLK, D_IN), lambda i: (i, 0)),
            pl.BlockSpec((PRE_BLK, RANK), lambda i: (i, 0)),
            pl.BlockSpec((PRE_BLK, RANK), lambda i: (i, 0)),
        ],
        out_shape=[
            jax.ShapeDtypeStruct((N, D_IN), jnp.float32),
            jax.ShapeDtypeStruct((N, RANK), jnp.float32),
            jax.ShapeDtypeStruct((N, RANK), jnp.float32),
        ],
    )(X, enc_W1, eb1, enc_W2, eb2, Ws, Wr, B)

    rowspec = pl.BlockSpec((BLK, 1), lambda i: (i, 0))
    rowshape = jax.ShapeDtypeStruct((N, 1), jnp.float32)
    h1, mrow, emin, srow = pl.pallas_call(
        _round1_kernel,
        grid=(N // BLK,),
        in_specs=[
            pl.BlockSpec((BLK, RANK), lambda i: (i, 0)),
            full((N, RANK)), full((N, D_IN)),
            full((D_IN, D_HIDDEN)), full((1, D_HIDDEN)),
        ],
        out_specs=[
            pl.BlockSpec((BLK, D_HIDDEN), lambda i: (i, 0)),
            rowspec, rowspec, rowspec,
        ],
        out_shape=[
            jax.ShapeDtypeStruct((N, D_HIDDEN), jnp.float32),
            rowshape, rowshape, rowshape,
        ],
    )(sb, r, xm, g_W1, gb1)

    y = pl.pallas_call(
        _round2_kernel,
        grid=(N // BLK,),
        in_specs=[
            pl.BlockSpec((BLK, RANK), lambda i: (i, 0)),
            full((N, RANK)), full((N, D_HIDDEN)),
            rowspec, rowspec, rowspec,
            full((D_HIDDEN, D_HIDDEN)), full((1, D_HIDDEN)),
            full((D_HIDDEN, 1)), full((1, 1)),
        ],
        out_specs=pl.BlockSpec((BLK, 1), lambda i: (i, 0)),
        out_shape=jax.ShapeDtypeStruct((N, 1), jnp.float32),
    )(sb, r, h1, mrow, emin, srow, g_W2, gb2, g_Wo, gbo)

    return y


# speedup vs baseline: 1.8703x; 1.0142x over previous
import numpy as np
import jax
import jax.numpy as jnp
from jax.experimental import pallas as pl

N = 4096
D_IN = 128
RANK = 16
D_HIDDEN = 128
BOTTLENECK = 64
TOPK = 32

PRE_BLK = 512
BLK = 512

_SIGN = np.int32(-2**31)
_MANT = np.int32(0x7FFFFFFF)


def _keys(l):
    bits = jax.lax.bitcast_convert_type(l, jnp.int32)
    return jnp.where(bits < 0, bits ^ _MANT, bits)


def _unkey(k):
    bits = jnp.where(k < 0, k ^ _MANT, k)
    return jax.lax.bitcast_convert_type(bits, jnp.float32)


def _prelude_kernel(x_ref, ew1_ref, eb1_ref, ew2_ref, eb2_ref, ws_ref, wr_ref,
                    b_ref, xm_ref, sb_ref, r_ref):
    x = x_ref[...]
    h = jnp.maximum(
        jnp.dot(x, ew1_ref[...], preferred_element_type=jnp.float32)
        + eb1_ref[...], 0.0)
    xm = x + jnp.dot(h, ew2_ref[...], preferred_element_type=jnp.float32) \
        + eb2_ref[...]
    xm_ref[...] = xm
    s = jax.nn.softmax(
        jnp.dot(xm, ws_ref[...], preferred_element_type=jnp.float32), axis=-1)
    r = jax.nn.softmax(
        jnp.dot(xm, wr_ref[...], preferred_element_type=jnp.float32), axis=-1)
    sb_ref[...] = jnp.dot(s, b_ref[...], preferred_element_type=jnp.float32)
    r_ref[...] = r


def _kth_key(vals, k, lo, hi):
    def cond(carry):
        lo_c, hi_c = carry
        return jnp.any(hi_c > lo_c)

    def step(carry):
        lo_c, hi_c = carry
        c = lo_c + jax.lax.shift_right_logical((hi_c - lo_c) + 1, 1)
        cnt = jnp.sum((vals >= _unkey(c)).astype(jnp.int32), axis=1,
                      keepdims=True)
        feas = cnt >= k
        return jnp.where(feas, c, lo_c), jnp.where(feas, hi_c, c - 1)

    def body(carry):
        for _ in range(4):
            carry = step(carry)
        return carry

    lo, hi = jax.lax.while_loop(cond, body, (lo, hi))
    return lo


def _bisect16(count_fn, k, lo, hi, unroll):
    def cond(carry):
        lo_c, hi_c = carry
        return jnp.any(hi_c > lo_c)

    def step(carry):
        lo_c, hi_c = carry
        c = lo_c + jax.lax.shift_right_logical((hi_c - lo_c) + 1, 1)
        feas = count_fn(c.astype(jnp.int16)) >= k
        return jnp.where(feas, c, lo_c), jnp.where(feas, hi_c, c - 1)

    def body(carry):
        for _ in range(unroll):
            carry = step(carry)
        return carry

    lo, hi = jax.lax.while_loop(cond, body, (lo, hi))
    return lo


def _sum16(x):
    w = x.shape[1]
    while w > 256:
        w //= 2
        x = x[:, :w] + x[:, w:2 * w]
    return jnp.sum(x.astype(jnp.int32), axis=1, keepdims=True)


def _kth_key_split(l, k, t0, hi1):
    bits = jax.lax.bitcast_convert_type(l, jnp.int32)
    hi_raw = jax.lax.shift_right_arithmetic(bits, 16).astype(jnp.int16)
    lo_raw = bits.astype(jnp.int16)
    neg = hi_raw < 0
    khi = jnp.where(neg, hi_raw ^ jnp.int16(0x7FFF), hi_raw)
    klo = jnp.where(neg, lo_raw ^ jnp.int16(0x7FFF), lo_raw ^ jnp.int16(-0x8000))
    one = jnp.int16(1)
    zero = jnp.int16(0)

    def cnt_hi(c16):
        return _sum16(jnp.where(khi >= c16, one, zero))

    t_hi = _bisect16(cnt_hi, k,
                     jax.lax.shift_right_arithmetic(t0, 16),
                     jax.lax.shift_right_arithmetic(hi1, 16), unroll=4)
    t_hi16 = t_hi.astype(jnp.int16)
    n_gt = _sum16(jnp.where(khi > t_hi16, one, zero))
    eq = jnp.where(khi == t_hi16, one, zero)

    def cnt_lo(c16):
        return n_gt + _sum16(jnp.where(klo >= c16, eq, zero))

    lo_c = jnp.full_like(t_hi, -32768)
    hi_c = jnp.full_like(t_hi, 32767)
    for _ in range(16):
        c = lo_c + jax.lax.shift_right_logical((hi_c - lo_c) + 1, 1)
        feas = cnt_lo(c.astype(jnp.int16)) >= k
        lo_c = jnp.where(feas, c, lo_c)
        hi_c = jnp.where(feas, hi_c, c - 1)
    c_lo = lo_c
    return jax.lax.shift_left(t_hi, 16) | \
        ((c_lo & jnp.int32(0xFFFF)) ^ jnp.int32(0x8000))


def _round1_kernel(sb_ref, r_ref, xm_ref, gw1_ref, gb1_ref,
                   h1_ref, m_ref, emin_ref, s_ref):
    sb = sb_ref[...]
    r = r_ref[...]
    l = jax.lax.dot_general(sb, r, (((1,), (1,)), ((), ())),
                            preferred_element_type=jnp.float32)

    m = jnp.max(l, axis=1, keepdims=True)
    lo1 = _keys(jnp.min(l, axis=1, keepdims=True))
    hi1 = _keys(m)
    t_star = _kth_key_split(l, TOPK, lo1, hi1)
    thr_f = _unkey(t_star)

    e = jnp.exp(l - m)
    z = jnp.sum(e, axis=1, keepdims=True)
    e_thr = jnp.exp(thr_f - m)
    a_thr = e_thr / z
    bits_t = jax.lax.bitcast_convert_type(e_thr, jnp.int32)
    e_min = e_thr
    for d in range(1, 9):
        cand = jax.lax.bitcast_convert_type(
            jnp.maximum(bits_t - d, 0), jnp.float32)
        ok = (cand / z) >= a_thr
        e_min = jnp.where(ok, cand, e_min)
    kept = e >= e_min
    ek = jnp.where(kept, e, 0.0)
    sume = jnp.sum(ek, axis=1, keepdims=True)
    rz = 1.0 / z
    den2 = jnp.maximum(sume * rz, 1e-8)
    s = rz * (1.0 / den2)
    w = ek * s
    msg = jnp.dot(w, xm_ref[...], preferred_element_type=jnp.float32)
    h1 = jnp.maximum(
        jnp.dot(msg, gw1_ref[...], preferred_element_type=jnp.float32)
        + gb1_ref[...], 0.0)
    h1_ref[...] = h1
    m_ref[...] = m
    emin_ref[...] = e_min
    s_ref[...] = s


def _round2_kernel(sb_ref, r_ref, h1_ref, m_ref, emin_ref, s_ref,
                   gw2_ref, gb2_ref, gwo_ref, gbo_ref, y_ref):
    sb = sb_ref[...]
    r = r_ref[...]
    l = jax.lax.dot_general(sb, r, (((1,), (1,)), ((), ())),
                            preferred_element_type=jnp.float32)
    e = jnp.exp(l - m_ref[...])
    w = jnp.where(e >= emin_ref[...], e, 0.0) * s_ref[...]
    msg = jnp.dot(w, h1_ref[...], preferred_element_type=jnp.float32)
    h2 = jnp.maximum(
        jnp.dot(msg, gw2_ref[...], preferred_element_type=jnp.float32)
        + gb2_ref[...], 0.0)
    y_ref[...] = jnp.dot(h2, gwo_ref[...], preferred_element_type=jnp.float32) \
        + gbo_ref[...]


def kernel(X, enc_W1, enc_b1, enc_W2, enc_b2, Ws, Wr, B,
           g_W1, g_b1, g_W2, g_b2, g_Wo, g_bo):
    eb1 = enc_b1.reshape(1, BOTTLENECK)
    eb2 = enc_b2.reshape(1, D_IN)
    gb1 = g_b1.reshape(1, D_HIDDEN)
    gb2 = g_b2.reshape(1, D_HIDDEN)
    gbo = g_bo.reshape(1, 1)

    full = lambda shape: pl.BlockSpec(shape, lambda i: (0, 0))

    xm, sb, r = pl.pallas_call(
        _prelude_kernel,
        grid=(N // PRE_BLK,),
        in_specs=[
            pl.BlockSpec((PRE_BLK, D_IN), lambda i: (i, 0)),
            full((D_IN, BOTTLENECK)), full((1, BOTTLENECK)),
            full((BOTTLENECK, D_IN)), full((1, D_IN)),
            full((D_IN, RANK)), full((D_IN, RANK)), full((RANK, RANK)),
        ],
        out_specs=[
            pl.BlockSpec((PRE_BLK, D_IN), lambda i: (i, 0)),
            pl.BlockSpec((PRE_BLK, RANK), lambda i: (i, 0)),
            pl.BlockSpec((PRE_BLK, RANK), lambda i: (i, 0)),
        ],
        out_shape=[
            jax.ShapeDtypeStruct((N, D_IN), jnp.float32),
            jax.ShapeDtypeStruct((N, RANK), jnp.float32),
            jax.ShapeDtypeStruct((N, RANK), jnp.float32),
        ],
    )(X, enc_W1, eb1, enc_W2, eb2, Ws, Wr, B)

    rowspec = pl.BlockSpec((BLK, 1), lambda i: (i, 0))
    rowshape = jax.ShapeDtypeStruct((N, 1), jnp.float32)
    h1, mrow, emin, srow = pl.pallas_call(
        _round1_kernel,
        grid=(N // BLK,),
        in_specs=[
            pl.BlockSpec((BLK, RANK), lambda i: (i, 0)),
            full((N, RANK)), full((N, D_IN)),
            full((D_IN, D_HIDDEN)), full((1, D_HIDDEN)),
        ],
        out_specs=[
            pl.BlockSpec((BLK, D_HIDDEN), lambda i: (i, 0)),
            rowspec, rowspec, rowspec,
        ],
        out_shape=[
            jax.ShapeDtypeStruct((N, D_HIDDEN), jnp.float32),
            rowshape, rowshape, rowshape,
        ],
    )(sb, r, xm, g_W1, gb1)

    y = pl.pallas_call(
        _round2_kernel,
        grid=(N // BLK,),
        in_specs=[
            pl.BlockSpec((BLK, RANK), lambda i: (i, 0)),
            full((N, RANK)), full((N, D_HIDDEN)),
            rowspec, rowspec, rowspec,
            full((D_HIDDEN, D_HIDDEN)), full((1, D_HIDDEN)),
            full((D_HIDDEN, 1)), full((1, 1)),
        ],
        out_specs=pl.BlockSpec((BLK, 1), lambda i: (i, 0)),
        out_shape=jax.ShapeDtypeStruct((N, 1), jnp.float32),
    )(sb, r, h1, mrow, emin, srow, g_W2, gb2, g_Wo, gbo)

    return y
